# Initial kernel scaffold; baseline (speedup 1.0000x reference)
import jax
import jax.numpy as jnp
from jax.experimental import pallas as pl


def kernel(x, attr, mask, W, bias):
    raise NotImplementedError("write your pallas kernel here")


# speedup vs baseline: 10.8280x; 10.8280x over previous
import functools

import jax
import jax.numpy as jnp
from jax.experimental import pallas as pl


def _reduce_body(x_ref, attr_ref, a_ref, *, num_patches, patch):
    b = pl.program_id(0)
    s = jnp.sign(x_ref[b])
    v = jnp.maximum(attr_ref[0, 0] * s[None, :], 0.0)
    psum = v.reshape(num_patches, patch, v.shape[-1]).sum(axis=(1, 2))
    a_ref[0, 0, 0] = psum


def _tail_body(a_ref, xr_ref, wt_ref, biasr_ref, out_ref, *, num_patches, patch):
    P = num_patches
    T = xr_ref[:] * wt_ref[:]
    N = T.shape[-1]
    n_iota = jax.lax.broadcasted_iota(jnp.int32, (N, P), 0)
    p_iota = jax.lax.broadcasted_iota(jnp.int32, (N, P), 1)
    ind = ((n_iota // patch) == p_iota).astype(jnp.float32)
    pd = jnp.dot(T, ind, preferred_element_type=jnp.float32)

    a2 = a_ref[:]
    ap = a2[:, :, None]
    aq = a2[:, None, :]
    qi = jax.lax.broadcasted_iota(jnp.int32, (a2.shape[0], P, P), 2)
    pi = jax.lax.broadcasted_iota(jnp.int32, (a2.shape[0], P, P), 1)
    beats = (aq > ap) | ((aq == ap) & (qi < pi))
    rank = jnp.sum(beats.astype(jnp.float32), axis=-1)
    wgt = jnp.float32(P) - rank

    S = jnp.sum(wgt * pd, axis=-1, keepdims=True)
    biasr = biasr_ref[:]
    inf0 = jnp.sum(pd, axis=-1, keepdims=True) + biasr
    dx = jnp.float32(1.0 / (P + 2))
    out_ref[:] = dx * (0.5 * (1.0 + biasr / inf0)
                       + (jnp.float32(P) * inf0 - S) / inf0)


def kernel(x, attr, mask, W, bias):
    B, M, F, N = attr.shape
    patch = int(F * 0.0625)
    P = F // patch

    a = pl.pallas_call(
        functools.partial(_reduce_body, num_patches=P, patch=patch),
        grid=(B, M),
        in_specs=[
            pl.BlockSpec((B, N), lambda b, m: (0, 0)),
            pl.BlockSpec((1, 1, F, N), lambda b, m: (b, m, 0, 0)),
        ],
        out_specs=pl.BlockSpec((1, 1, 1, P), lambda b, m: (b, m, 0, 0)),
        out_shape=jax.ShapeDtypeStruct((B, M, 1, P), jnp.float32),
    )(x, attr)

    a2 = a.reshape(B * M, P)
    xr = jnp.repeat(x, M, axis=0)
    wt = jnp.tile(W.T, (B, 1))
    biasr = jnp.tile(bias, B).reshape(B * M, 1)

    out_flat = pl.pallas_call(
        functools.partial(_tail_body, num_patches=P, patch=patch),
        out_shape=jax.ShapeDtypeStruct((B * M, 1), jnp.float32),
    )(a2, xr, wt, biasr)
    return out_flat.reshape(B, M)


# speedup vs baseline: 12.5354x; 1.1577x over previous
import functools

import jax
import jax.numpy as jnp
from jax.experimental import pallas as pl


def _reduce_body(x_ref, attr_ref, a_ref, *, num_patches, patch, m_blk):
    b = pl.program_id(0)
    s = jnp.sign(x_ref[b])
    for j in range(m_blk):
        v = jnp.maximum(attr_ref[0, j] * s[None, :], 0.0)
        psum = v.reshape(num_patches, patch, v.shape[-1]).sum(axis=(1, 2))
        a_ref[0, j, 0] = psum


def _tail_body(a_ref, xr_ref, wt_ref, biasr_ref, out_ref, *, num_patches, patch):
    P = num_patches
    T = xr_ref[:] * wt_ref[:]
    N = T.shape[-1]
    n_iota = jax.lax.broadcasted_iota(jnp.int32, (N, P), 0)
    p_iota = jax.lax.broadcasted_iota(jnp.int32, (N, P), 1)
    ind = ((n_iota // patch) == p_iota).astype(jnp.float32)
    pd = jnp.dot(T, ind, preferred_element_type=jnp.float32)

    a2 = a_ref[:]
    ap = a2[:, :, None]
    aq = a2[:, None, :]
    qi = jax.lax.broadcasted_iota(jnp.int32, (a2.shape[0], P, P), 2)
    pi = jax.lax.broadcasted_iota(jnp.int32, (a2.shape[0], P, P), 1)
    beats = (aq > ap) | ((aq == ap) & (qi < pi))
    rank = jnp.sum(beats.astype(jnp.float32), axis=-1)
    wgt = jnp.float32(P) - rank

    S = jnp.sum(wgt * pd, axis=-1, keepdims=True)
    biasr = biasr_ref[:]
    inf0 = jnp.sum(pd, axis=-1, keepdims=True) + biasr
    dx = jnp.float32(1.0 / (P + 2))
    out_ref[:] = dx * (0.5 * (1.0 + biasr / inf0)
                       + (jnp.float32(P) * inf0 - S) / inf0)


def kernel(x, attr, mask, W, bias):
    B, M, F, N = attr.shape
    patch = int(F * 0.0625)
    P = F // patch

    M_BLK = 2
    a = pl.pallas_call(
        functools.partial(_reduce_body, num_patches=P, patch=patch, m_blk=M_BLK),
        grid=(B, M // M_BLK),
        in_specs=[
            pl.BlockSpec((B, N), lambda b, m: (0, 0)),
            pl.BlockSpec((1, M_BLK, F, N), lambda b, m: (b, m, 0, 0)),
        ],
        out_specs=pl.BlockSpec((1, M_BLK, 1, P), lambda b, m: (b, m, 0, 0)),
        out_shape=jax.ShapeDtypeStruct((B, M, 1, P), jnp.float32),
    )(x, attr)

    a2 = a.reshape(B * M, P)
    xr = jnp.repeat(x, M, axis=0)
    wt = jnp.tile(W.T, (B, 1))
    biasr = jnp.tile(bias, B).reshape(B * M, 1)

    out_flat = pl.pallas_call(
        functools.partial(_tail_body, num_patches=P, patch=patch),
        out_shape=jax.ShapeDtypeStruct((B * M, 1), jnp.float32),
    )(a2, xr, wt, biasr)
    return out_flat.reshape(B, M)
